# baseline (speedup 1.0000x reference)
import functools

import jax
import jax.numpy as jnp
from jax import lax
from jax.experimental import pallas as pl
from jax.experimental.pallas import tpu as pltpu
from jax.experimental.pallas import tpu_sc as plsc

N_PAIR = 32
N_CTX = 8
MAX_L = 40
D = 768
NUM_BASE = 92
NUM_NOVEL = 40
N_CLS = NUM_BASE + NUM_NOVEL
VOCAB = 1000
D_ENTI = 256
SUF_L = MAX_L - 1 - N_CTX

C_BLK = 66
NCC = N_CLS // C_BLK

N_IDS = 2 * N_PAIR
G_WORKERS = 8
G_ROWS = N_IDS // G_WORKERS
TM_CHUNK = N_CLS * MAX_L

TB = 8
N_TB = MAX_L // TB

_SC_MESH = plsc.VectorSubcoreMesh(core_axis_name="c", subcore_axis_name="s")


@functools.partial(
    pl.kernel,
    mesh=_SC_MESH,
    out_type=jax.ShapeDtypeStruct((N_IDS, D_ENTI), jnp.float32),
    scratch_types=[
        pltpu.VMEM((G_ROWS,), jnp.int32),
        pltpu.VMEM((G_ROWS, D_ENTI), jnp.float32),
        pltpu.SemaphoreType.DMA,
    ],
)
def _sc_gather(ids_hbm, enti_hbm, rows_out, idx_v, rows_v, sem):
    wid = lax.axis_index("s") * 2 + lax.axis_index("c")

    @pl.when(wid < G_WORKERS)
    def _():
        base = wid * G_ROWS
        pltpu.sync_copy(ids_hbm.at[pl.ds(base, G_ROWS)], idx_v)
        pltpu.async_copy(enti_hbm.at[idx_v], rows_v, sem).wait()
        pltpu.sync_copy(rows_v, rows_out.at[pl.ds(base, G_ROWS)])


@functools.partial(
    pl.kernel,
    mesh=_SC_MESH,
    out_type=jax.ShapeDtypeStruct((N_PAIR * TM_CHUNK,), jnp.int32),
    scratch_types=[pltpu.VMEM((TM_CHUNK,), jnp.int32)],
)
def _sc_mask(tm_hbm, tm_out, tm_v):
    wid = lax.axis_index("s") * 2 + lax.axis_index("c")
    pltpu.sync_copy(tm_hbm, tm_v)
    pltpu.sync_copy(tm_v, tm_out.at[pl.ds(wid * TM_CHUNK, TM_CHUNK)])


def _suffix_body(suf_ref, subj_ref, obj_ref):
    t = pl.program_id(1)
    for tt in range(N_TB - 2):
        @pl.when(t == tt)
        def _():
            suf = suf_ref[:, TB - 1 + TB * tt:2 * TB - 1 + TB * tt, :]
            subj_ref[:] = suf
            obj_ref[:] = suf


def _head_body(g_ref, sW1_ref, sb1_ref, sW2_ref, oW1_ref, ob1_ref,
               oW2_ref, sctx_in_ref, octx_in_ref, pre_ref, suf_ref,
               subj_in, obj_in, subj_ref, obj_ref, sctx_scr, octx_scr):
    del subj_in, obj_in
    cc = pl.program_id(0)
    t = pl.program_id(1)
    p = pl.program_id(2)

    @pl.when((cc == 0) & (t == 0) & (p == 0))
    def _():
        s_e = g_ref[0:N_PAIR]
        o_e = g_ref[N_PAIR:N_IDS]
        s_h = jnp.maximum(
            jnp.dot(s_e, sW1_ref[:], preferred_element_type=jnp.float32)
            + sb1_ref[:], 0.0)
        o_h = jnp.maximum(
            jnp.dot(o_e, oW1_ref[:], preferred_element_type=jnp.float32)
            + ob1_ref[:], 0.0)
        s_emb = jnp.dot(s_h, sW2_ref[:], preferred_element_type=jnp.float32)
        o_emb = jnp.dot(o_h, oW2_ref[:], preferred_element_type=jnp.float32)
        sctx_scr[:] = sctx_in_ref[:][None, :, :] + s_emb[:, None, :]
        octx_scr[:] = octx_in_ref[:][None, :, :] + o_emb[:, None, :]

    s_ctx = sctx_scr[p]
    o_ctx = octx_scr[p]

    @pl.when(t == 0)
    def _():
        subj_ref[:, 0:1, :] = pre_ref[:]
        subj_ref[:, 1:TB, :] = jnp.broadcast_to(
            s_ctx[None, 0:TB - 1, :], (C_BLK, TB - 1, D))
        obj_ref[:, 0:1, :] = pre_ref[:]
        obj_ref[:, 1:TB, :] = jnp.broadcast_to(
            o_ctx[None, 0:TB - 1, :], (C_BLK, TB - 1, D))

    @pl.when(t == 1)
    def _():
        suf = suf_ref[:, 0:TB - 1, :]
        subj_ref[:, 0:1, :] = jnp.broadcast_to(
            s_ctx[None, TB - 1:TB, :], (C_BLK, 1, D))
        subj_ref[:, 1:TB, :] = suf
        obj_ref[:, 0:1, :] = jnp.broadcast_to(
            o_ctx[None, TB - 1:TB, :], (C_BLK, 1, D))
        obj_ref[:, 1:TB, :] = suf


def kernel(so_cls_ids, enti_txt_embds, prefix_embds, suffix_embds, token_mask,
           subj_ctx_embds, obj_ctx_embds, sW1, sb1, sW2, oW1, ob1, oW2):
    prefix_sl = prefix_embds[1:N_CLS + 1]
    suffix_sl = suffix_embds[1:N_CLS + 1]
    tm_flat = token_mask[1:N_CLS + 1].reshape(-1)
    ids_flat = so_cls_ids.T.reshape(-1)

    gathered = _sc_gather(ids_flat, enti_txt_embds)
    tm_rep_flat = _sc_mask(tm_flat)

    out_sds = jax.ShapeDtypeStruct((N_PAIR * N_CLS, MAX_L, D), jnp.float32)

    subj1, obj1 = pl.pallas_call(
        _suffix_body,
        grid=(NCC, N_TB - 2, N_PAIR),
        in_specs=[
            pl.BlockSpec((C_BLK, SUF_L, D), lambda cc, t, p: (cc, 0, 0)),
        ],
        out_specs=[
            pl.BlockSpec((C_BLK, TB, D), lambda cc, t, p: (p * NCC + cc, t + 2, 0)),
            pl.BlockSpec((C_BLK, TB, D), lambda cc, t, p: (p * NCC + cc, t + 2, 0)),
        ],
        out_shape=(out_sds, out_sds),
    )(suffix_sl)

    const2 = lambda cc, t, p: (0, 0)
    subj, obj = pl.pallas_call(
        _head_body,
        grid=(NCC, 2, N_PAIR),
        in_specs=[
            pl.BlockSpec((N_IDS, D_ENTI), const2),
            pl.BlockSpec((D_ENTI, D_ENTI), const2),
            pl.BlockSpec((D_ENTI,), lambda cc, t, p: (0,)),
            pl.BlockSpec((D_ENTI, D), const2),
            pl.BlockSpec((D_ENTI, D_ENTI), const2),
            pl.BlockSpec((D_ENTI,), lambda cc, t, p: (0,)),
            pl.BlockSpec((D_ENTI, D), const2),
            pl.BlockSpec((N_CTX, D), const2),
            pl.BlockSpec((N_CTX, D), const2),
            pl.BlockSpec((C_BLK, 1, D), lambda cc, t, p: (cc, 0, 0)),
            pl.BlockSpec((C_BLK, TB, D), lambda cc, t, p: (cc, 0, 0)),
            pl.BlockSpec(memory_space=pl.ANY),
            pl.BlockSpec(memory_space=pl.ANY),
        ],
        out_specs=[
            pl.BlockSpec((C_BLK, TB, D), lambda cc, t, p: (p * NCC + cc, t, 0)),
            pl.BlockSpec((C_BLK, TB, D), lambda cc, t, p: (p * NCC + cc, t, 0)),
        ],
        out_shape=(out_sds, out_sds),
        scratch_shapes=[
            pltpu.VMEM((N_PAIR, N_CTX, D), jnp.float32),
            pltpu.VMEM((N_PAIR, N_CTX, D), jnp.float32),
        ],
        input_output_aliases={11: 0, 12: 1},
    )(gathered, sW1, sb1, sW2, oW1, ob1, oW2, subj_ctx_embds, obj_ctx_embds,
      prefix_sl, suffix_sl, subj1, obj1)

    return subj, obj, tm_rep_flat.reshape(N_PAIR * N_CLS, MAX_L)


# speedup vs baseline: 1.0961x; 1.0961x over previous
import functools

import jax
import jax.numpy as jnp
from jax import lax
from jax.experimental import pallas as pl
from jax.experimental.pallas import tpu as pltpu
from jax.experimental.pallas import tpu_sc as plsc

N_PAIR = 32
N_CTX = 8
MAX_L = 40
D = 768
NUM_BASE = 92
NUM_NOVEL = 40
N_CLS = NUM_BASE + NUM_NOVEL
VOCAB = 1000
D_ENTI = 256
SUF_L = MAX_L - 1 - N_CTX

C_BLK = 66
NCC = N_CLS // C_BLK

N_IDS = 2 * N_PAIR
G_WORKERS = 8
G_ROWS = N_IDS // G_WORKERS
TM_CHUNK = N_CLS * MAX_L

_SC_MESH = plsc.VectorSubcoreMesh(core_axis_name="c", subcore_axis_name="s")


@functools.partial(
    pl.kernel,
    mesh=_SC_MESH,
    out_type=jax.ShapeDtypeStruct((N_IDS, D_ENTI), jnp.float32),
    scratch_types=[
        pltpu.VMEM((G_ROWS,), jnp.int32),
        pltpu.VMEM((G_ROWS, D_ENTI), jnp.float32),
        pltpu.SemaphoreType.DMA,
    ],
)
def _sc_gather(ids_hbm, enti_hbm, rows_out, idx_v, rows_v, sem):
    wid = lax.axis_index("s") * 2 + lax.axis_index("c")

    @pl.when(wid < G_WORKERS)
    def _():
        base = wid * G_ROWS
        pltpu.sync_copy(ids_hbm.at[pl.ds(base, G_ROWS)], idx_v)
        pltpu.async_copy(enti_hbm.at[idx_v], rows_v, sem).wait()
        pltpu.sync_copy(rows_v, rows_out.at[pl.ds(base, G_ROWS)])


@functools.partial(
    pl.kernel,
    mesh=_SC_MESH,
    out_type=jax.ShapeDtypeStruct((N_PAIR * TM_CHUNK,), jnp.int32),
    scratch_types=[pltpu.VMEM((TM_CHUNK,), jnp.int32)],
)
def _sc_mask(tm_hbm, tm_out, tm_v):
    wid = lax.axis_index("s") * 2 + lax.axis_index("c")
    pltpu.sync_copy(tm_hbm, tm_v)
    pltpu.sync_copy(tm_v, tm_out.at[pl.ds(wid * TM_CHUNK, TM_CHUNK)])


def _assemble_body(g_ref, sW1_ref, sb1_ref, sW2_ref, oW1_ref, ob1_ref,
                   oW2_ref, sctx_in_ref, octx_in_ref,
                   pre_ref, suf_ref, subj_ref, obj_ref,
                   sctx_scr, octx_scr):
    cc = pl.program_id(0)
    p = pl.program_id(1)

    @pl.when((cc == 0) & (p == 0))
    def _():
        s_e = g_ref[0:N_PAIR]
        o_e = g_ref[N_PAIR:N_IDS]
        s_h = jnp.maximum(
            jnp.dot(s_e, sW1_ref[:], preferred_element_type=jnp.float32)
            + sb1_ref[:], 0.0)
        o_h = jnp.maximum(
            jnp.dot(o_e, oW1_ref[:], preferred_element_type=jnp.float32)
            + ob1_ref[:], 0.0)
        s_emb = jnp.dot(s_h, sW2_ref[:], preferred_element_type=jnp.float32)
        o_emb = jnp.dot(o_h, oW2_ref[:], preferred_element_type=jnp.float32)
        sctx_scr[:] = sctx_in_ref[:][None, :, :] + s_emb[:, None, :]
        octx_scr[:] = octx_in_ref[:][None, :, :] + o_emb[:, None, :]

    s_ctx = jnp.broadcast_to(sctx_scr[p][None, :, :], (C_BLK, N_CTX, D))
    o_ctx = jnp.broadcast_to(octx_scr[p][None, :, :], (C_BLK, N_CTX, D))
    pre = pre_ref[:]
    suf = suf_ref[:]
    subj_ref[:, 0:1, :] = pre
    subj_ref[:, 1:1 + N_CTX, :] = s_ctx
    subj_ref[:, 1 + N_CTX:MAX_L, :] = suf
    obj_ref[:, 0:1, :] = pre
    obj_ref[:, 1:1 + N_CTX, :] = o_ctx
    obj_ref[:, 1 + N_CTX:MAX_L, :] = suf


def kernel(so_cls_ids, enti_txt_embds, prefix_embds, suffix_embds, token_mask,
           subj_ctx_embds, obj_ctx_embds, sW1, sb1, sW2, oW1, ob1, oW2):
    prefix_sl = prefix_embds[1:N_CLS + 1]
    suffix_sl = suffix_embds[1:N_CLS + 1]
    tm_flat = token_mask[1:N_CLS + 1].reshape(-1)
    ids_flat = so_cls_ids.T.reshape(-1)

    gathered = _sc_gather(ids_flat, enti_txt_embds)
    tm_rep_flat = _sc_mask(tm_flat)

    const2 = lambda cc, p: (0, 0)
    subj, obj = pl.pallas_call(
        _assemble_body,
        grid=(NCC, N_PAIR),
        in_specs=[
            pl.BlockSpec((N_IDS, D_ENTI), const2),
            pl.BlockSpec((D_ENTI, D_ENTI), const2),
            pl.BlockSpec((D_ENTI,), lambda cc, p: (0,)),
            pl.BlockSpec((D_ENTI, D), const2),
            pl.BlockSpec((D_ENTI, D_ENTI), const2),
            pl.BlockSpec((D_ENTI,), lambda cc, p: (0,)),
            pl.BlockSpec((D_ENTI, D), const2),
            pl.BlockSpec((N_CTX, D), const2),
            pl.BlockSpec((N_CTX, D), const2),
            pl.BlockSpec((C_BLK, 1, D), lambda cc, p: (cc, 0, 0)),
            pl.BlockSpec((C_BLK, SUF_L, D), lambda cc, p: (cc, 0, 0)),
        ],
        out_specs=[
            pl.BlockSpec((C_BLK, MAX_L, D), lambda cc, p: (p * NCC + cc, 0, 0)),
            pl.BlockSpec((C_BLK, MAX_L, D), lambda cc, p: (p * NCC + cc, 0, 0)),
        ],
        out_shape=(
            jax.ShapeDtypeStruct((N_PAIR * N_CLS, MAX_L, D), jnp.float32),
            jax.ShapeDtypeStruct((N_PAIR * N_CLS, MAX_L, D), jnp.float32),
        ),
        scratch_shapes=[
            pltpu.VMEM((N_PAIR, N_CTX, D), jnp.float32),
            pltpu.VMEM((N_PAIR, N_CTX, D), jnp.float32),
        ],
    )(gathered, sW1, sb1, sW2, oW1, ob1, oW2, subj_ctx_embds, obj_ctx_embds,
      prefix_sl, suffix_sl)

    return subj, obj, tm_rep_flat.reshape(N_PAIR * N_CLS, MAX_L)
